# baseline (speedup 1.0000x reference)
import jax
import jax.numpy as jnp
import numpy as np
from jax import lax
from jax.experimental import pallas as pl

ALPHA = 0.5
BETA = 0.3
T = 0.5
IMG_H = 1024.0
IMG_W = 1024.0
N_PROP = 20000
N_GT = 64
N_PER = 4
N_GEN = N_GT * N_PER
N_OUT = N_PROP + N_GEN

_LO = float(np.log(1.0 - BETA))
_HI = float(np.log(1.0 + BETA))


def _iota2(shape, dim):
    return lax.broadcasted_iota(jnp.int32, shape, dim)


def _split3(v):
    b1 = v.astype(jnp.bfloat16)
    r1 = v - b1.astype(jnp.float32)
    b2 = r1.astype(jnp.bfloat16)
    b3 = (r1 - b2.astype(jnp.float32)).astype(jnp.bfloat16)
    return b1, b2, b3


def _body(x_ref, lg_ref, gtb_ref, gt4_ref, u4_ref, A_ref, p128_ref, p640_ref, out_ref, gen_ref):
    f32 = jnp.float32
    A = A_ref[...]
    P128 = p128_ref[...]
    P640 = p640_ref[...]

    def dotb(a, b):
        return lax.dot_general(
            a, b, (((1,), (0,)), ((), ())), preferred_element_type=f32
        )

    def dot_pl(p, v):
        b1, b2, b3 = _split3(v)
        return (dotb(p, b1) + dotb(p, b2)) + dotb(p, b3)

    def dot_pr(v, p):
        b1, b2, b3 = _split3(v)
        return (dotb(b1, p) + dotb(b2, p)) + dotb(b3, p)

    x = jnp.concatenate([x_ref[...], jnp.zeros((15, 128), f32)], axis=0)
    w = dot_pl(A, dot_pr(x, P128)).reshape(4, 160, 128)
    comps = [
        jnp.concatenate([w[q][:, 32 * c : 32 * c + 32] for q in range(4)], axis=1)
        for c in range(4)
    ]
    px1, py1, px2, py2 = comps
    zlg = lg_ref[...]

    gx1 = gtb_ref[:, 0:1]
    gy1 = gtb_ref[:, 1:2]
    gx2 = gtb_ref[:, 2:3]
    gy2 = gtb_ref[:, 3:4]
    ga = (gx2 - gx1) * (gy2 - gy1)

    def keep_mask(a1, b1, a2, b2, area_b):
        wx = jnp.maximum(jnp.minimum(gx2, a2) - jnp.maximum(gx1, a1), 0.0)
        wy = jnp.maximum(jnp.minimum(gy2, b2) - jnp.maximum(gy1, b1), 0.0)
        inter = wx * wy
        union = (ga + area_b) - inter
        iou = inter / jnp.maximum(union, 1e-9)
        m = jnp.max(iou, axis=0, keepdims=True)
        return jnp.where(m > T, 1.0, 0.0).astype(f32)

    krows = []
    for r in range(160):
        a1 = px1[r : r + 1, :]
        b1 = py1[r : r + 1, :]
        a2 = px2[r : r + 1, :]
        b2 = py2[r : r + 1, :]
        area_b = (a2 - a1) * (b2 - b1)
        krows.append(keep_mask(a1, b1, a2, b2, area_b))
    K = jnp.concatenate(krows, axis=0)

    zcat = jnp.concatenate(
        [px1 * K, py1 * K, px2 * K, py2 * K, zlg * K], axis=1
    )
    out_ref[...] = dot_pr(zcat, P640)

    gg_x1 = gt4_ref[0:1, :]
    gg_y1 = gt4_ref[1:2, :]
    gg_x2 = gt4_ref[2:3, :]
    gg_y2 = gt4_ref[3:4, :]
    gw = gg_x2 - gg_x1
    gh = gg_y2 - gg_y1
    tx = (u4_ref[0:1, :] - 0.5) * 2.0 * ALPHA
    ty = (u4_ref[1:2, :] - 0.5) * 2.0 * ALPHA
    tw = _LO + u4_ref[2:3, :] * (_HI - _LO)
    th = _LO + u4_ref[3:4, :] * (_HI - _LO)
    nx = gg_x1 + gw * tx
    ny = gg_y1 + gh * ty
    nw = gw * jnp.exp(tw)
    nh = gh * jnp.exp(th)
    x1 = jnp.minimum(jnp.maximum(nx, 0.0), IMG_W)
    y1 = jnp.minimum(jnp.maximum(ny, 0.0), IMG_H)
    x2 = jnp.minimum(jnp.maximum(nx + nw, 0.0), IMG_W)
    y2 = jnp.minimum(jnp.maximum(ny + nh, 0.0), IMG_H)
    area_b2 = (x2 - x1) * (y2 - y1)
    k2 = keep_mask(x1, y1, x2, y2, area_b2)
    zg = jnp.concatenate(
        [
            jnp.concatenate([v[:, 0:128], v[:, 128:256]], axis=0)
            for v in (x1 * k2, y1 * k2, x2 * k2, y2 * k2, k2)
        ],
        axis=1,
    )
    gen_ref[...] = dot_pr(zg, P640)


def _perm_consts():
    i = np.arange(640)
    A = (i[None, :] == 4 * (i[:, None] % 160) + i[:, None] // 160).astype(np.float32)
    l = np.arange(128)
    P128 = (l[:, None] == 4 * (l[None, :] % 32) + l[None, :] // 32).astype(np.float32)
    P640 = (i[None, :] == 5 * (i[:, None] % 128) + i[:, None] // 128).astype(np.float32)
    import ml_dtypes

    bf = ml_dtypes.bfloat16
    return A.astype(bf), P128.astype(bf), P640.astype(bf)


_A_NP, _P128_NP, _P640_NP = _perm_consts()


@jax.jit
def kernel(proposal_boxes, objectness_logits, gt_boxes, u_tx, u_ty, u_tw, u_th):
    xp = proposal_boxes.reshape(625, 128)
    lg = jnp.pad(objectness_logits, (0, 480)).reshape(160, 128)
    gt4 = jnp.repeat(gt_boxes.T, N_PER, axis=1)
    u4 = jnp.stack(
        [u_tx.T.reshape(-1), u_ty.T.reshape(-1), u_tw.T.reshape(-1), u_th.T.reshape(-1)]
    )
    out, gen = pl.pallas_call(
        _body,
        out_shape=[
            jax.ShapeDtypeStruct((160, 640), jnp.float32),
            jax.ShapeDtypeStruct((2, 640), jnp.float32),
        ],
    )(xp, lg, gt_boxes, gt4, u4, _A_NP, _P128_NP, _P640_NP)
    flat = jnp.concatenate([out.reshape(-1)[: N_PROP * 5], gen.reshape(-1)])
    return flat.reshape(N_OUT, 5)


# speedup vs baseline: 1.0068x; 1.0068x over previous
import jax
import jax.numpy as jnp
import numpy as np
from jax import lax
from jax.experimental import pallas as pl

ALPHA = 0.5
BETA = 0.3
T = 0.5
IMG_H = 1024.0
IMG_W = 1024.0
N_PROP = 20000
N_GT = 64
N_PER = 4
N_GEN = N_GT * N_PER
N_OUT = N_PROP + N_GEN

_LO = float(np.log(1.0 - BETA))
_HI = float(np.log(1.0 + BETA))


def _iota2(shape, dim):
    return lax.broadcasted_iota(jnp.int32, shape, dim)


def _split3(v):
    b1 = v.astype(jnp.bfloat16)
    r1 = v - b1.astype(jnp.float32)
    b2 = r1.astype(jnp.bfloat16)
    b3 = (r1 - b2.astype(jnp.float32)).astype(jnp.bfloat16)
    return b1, b2, b3


def _body(x_ref, lg_ref, gtb_ref, gt4_ref, u4_ref, out_ref, gen_ref):
    f32 = jnp.float32
    bf16 = jnp.bfloat16
    ai = _iota2((640, 640), 0)
    ar = _iota2((640, 640), 1)
    A = jnp.where(ar == 4 * (ai % 160) + ai // 160, 1.0, 0.0).astype(bf16)
    P640 = jnp.where(ar == 5 * (ai % 128) + ai // 128, 1.0, 0.0).astype(bf16)
    pl_ = _iota2((128, 128), 0)
    pm = _iota2((128, 128), 1)
    P128 = jnp.where(pl_ == 4 * (pm % 32) + pm // 32, 1.0, 0.0).astype(bf16)

    def dotb(a, b):
        return lax.dot_general(
            a, b, (((1,), (0,)), ((), ())), preferred_element_type=f32
        )

    def dot_pl(p, v):
        b1, b2, b3 = _split3(v)
        return (dotb(p, b1) + dotb(p, b2)) + dotb(p, b3)

    def dot_pr(v, p):
        b1, b2, b3 = _split3(v)
        return (dotb(b1, p) + dotb(b2, p)) + dotb(b3, p)

    x = jnp.concatenate([x_ref[...], jnp.zeros((15, 128), f32)], axis=0)
    w = dot_pl(A, dot_pr(x, P128)).reshape(4, 160, 128)
    comps = [
        jnp.concatenate([w[q][:, 32 * c : 32 * c + 32] for q in range(4)], axis=1)
        for c in range(4)
    ]
    px1, py1, px2, py2 = comps
    zlg = lg_ref[...]

    gx1 = gtb_ref[:, 0:1]
    gy1 = gtb_ref[:, 1:2]
    gx2 = gtb_ref[:, 2:3]
    gy2 = gtb_ref[:, 3:4]
    ga = (gx2 - gx1) * (gy2 - gy1)

    def keep_mask(a1, b1, a2, b2, area_b):
        wx = jnp.maximum(jnp.minimum(gx2, a2) - jnp.maximum(gx1, a1), 0.0)
        wy = jnp.maximum(jnp.minimum(gy2, b2) - jnp.maximum(gy1, b1), 0.0)
        inter = wx * wy
        union = (ga + area_b) - inter
        iou = inter / jnp.maximum(union, 1e-9)
        m = jnp.max(iou, axis=0, keepdims=True)
        return jnp.where(m > T, 1.0, 0.0).astype(f32)

    krows = []
    for r in range(160):
        a1 = px1[r : r + 1, :]
        b1 = py1[r : r + 1, :]
        a2 = px2[r : r + 1, :]
        b2 = py2[r : r + 1, :]
        area_b = (a2 - a1) * (b2 - b1)
        krows.append(keep_mask(a1, b1, a2, b2, area_b))
    K = jnp.concatenate(krows, axis=0)

    zcat = jnp.concatenate(
        [px1 * K, py1 * K, px2 * K, py2 * K, zlg * K], axis=1
    )
    out_ref[...] = dot_pr(zcat, P640)

    gg_x1 = gt4_ref[0:1, :]
    gg_y1 = gt4_ref[1:2, :]
    gg_x2 = gt4_ref[2:3, :]
    gg_y2 = gt4_ref[3:4, :]
    gw = gg_x2 - gg_x1
    gh = gg_y2 - gg_y1
    tx = (u4_ref[0:1, :] - 0.5) * 2.0 * ALPHA
    ty = (u4_ref[1:2, :] - 0.5) * 2.0 * ALPHA
    tw = _LO + u4_ref[2:3, :] * (_HI - _LO)
    th = _LO + u4_ref[3:4, :] * (_HI - _LO)
    nx = gg_x1 + gw * tx
    ny = gg_y1 + gh * ty
    nw = gw * jnp.exp(tw)
    nh = gh * jnp.exp(th)
    x1 = jnp.minimum(jnp.maximum(nx, 0.0), IMG_W)
    y1 = jnp.minimum(jnp.maximum(ny, 0.0), IMG_H)
    x2 = jnp.minimum(jnp.maximum(nx + nw, 0.0), IMG_W)
    y2 = jnp.minimum(jnp.maximum(ny + nh, 0.0), IMG_H)
    area_b2 = (x2 - x1) * (y2 - y1)
    k2 = keep_mask(x1, y1, x2, y2, area_b2)
    zg = jnp.concatenate(
        [
            jnp.concatenate([v[:, 0:128], v[:, 128:256]], axis=0)
            for v in (x1 * k2, y1 * k2, x2 * k2, y2 * k2, k2)
        ],
        axis=1,
    )
    gen_ref[...] = dot_pr(zg, P640)


@jax.jit
def kernel(proposal_boxes, objectness_logits, gt_boxes, u_tx, u_ty, u_tw, u_th):
    xp = proposal_boxes.reshape(625, 128)
    lg = jnp.pad(objectness_logits, (0, 480)).reshape(160, 128)
    gt4 = jnp.repeat(gt_boxes.T, N_PER, axis=1)
    u4 = jnp.stack(
        [u_tx.T.reshape(-1), u_ty.T.reshape(-1), u_tw.T.reshape(-1), u_th.T.reshape(-1)]
    )
    out, gen = pl.pallas_call(
        _body,
        out_shape=[
            jax.ShapeDtypeStruct((160, 640), jnp.float32),
            jax.ShapeDtypeStruct((2, 640), jnp.float32),
        ],
    )(xp, lg, gt_boxes, gt4, u4)
    flat = jnp.concatenate([out.reshape(-1)[: N_PROP * 5], gen.reshape(-1)])
    return flat.reshape(N_OUT, 5)


# speedup vs baseline: 4.5084x; 4.4778x over previous
import jax
import jax.numpy as jnp
import numpy as np
from jax.experimental import pallas as pl

ALPHA = 0.5
BETA = 0.3
T = 0.5
IMG_H = 1024.0
IMG_W = 1024.0
N_PROP = 20000
N_GT = 64
N_PER = 4
N_GEN = N_GT * N_PER
N_OUT = N_PROP + N_GEN
NPAD = 20480
CHUNK = 512

_LO = float(np.log(1.0 - BETA))
_HI = float(np.log(1.0 + BETA))


def _body(pbT, gtb, g8, main, gen):
    gx1 = gtb[:, 0:1]
    gy1 = gtb[:, 1:2]
    gx2 = gtb[:, 2:3]
    gy2 = gtb[:, 3:4]
    ga = (gx2 - gx1) * (gy2 - gy1)

    def keep_mask(a1, b1, a2, b2, area_b):
        wx = jnp.maximum(jnp.minimum(gx2, a2) - jnp.maximum(gx1, a1), 0.0)
        wy = jnp.maximum(jnp.minimum(gy2, b2) - jnp.maximum(gy1, b1), 0.0)
        inter = wx * wy
        union = (ga + area_b) - inter
        iou = inter / jnp.maximum(union, 1e-9)
        m = jnp.max(iou, axis=0, keepdims=True)
        return jnp.where(m > T, 1.0, 0.0).astype(jnp.float32)

    for c in range(NPAD // CHUNK):
        s = c * CHUNK
        px1 = pbT[0:1, s : s + CHUNK]
        py1 = pbT[1:2, s : s + CHUNK]
        px2 = pbT[2:3, s : s + CHUNK]
        py2 = pbT[3:4, s : s + CHUNK]
        area_b = (px2 - px1) * (py2 - py1)
        k = keep_mask(px1, py1, px2, py2, area_b)
        main[0:1, s : s + CHUNK] = px1 * k
        main[1:2, s : s + CHUNK] = py1 * k
        main[2:3, s : s + CHUNK] = px2 * k
        main[3:4, s : s + CHUNK] = py2 * k
        main[4:5, s : s + CHUNK] = pbT[4:5, s : s + CHUNK] * k

    gg_x1 = g8[0:1, :]
    gg_y1 = g8[1:2, :]
    gg_x2 = g8[2:3, :]
    gg_y2 = g8[3:4, :]
    gw = gg_x2 - gg_x1
    gh = gg_y2 - gg_y1
    tx = (g8[4:5, :] - 0.5) * 2.0 * ALPHA
    ty = (g8[5:6, :] - 0.5) * 2.0 * ALPHA
    tw = _LO + g8[6:7, :] * (_HI - _LO)
    th = _LO + g8[7:8, :] * (_HI - _LO)
    nx = gg_x1 + gw * tx
    ny = gg_y1 + gh * ty
    nw = gw * jnp.exp(tw)
    nh = gh * jnp.exp(th)
    x1 = jnp.minimum(jnp.maximum(nx, 0.0), IMG_W)
    y1 = jnp.minimum(jnp.maximum(ny, 0.0), IMG_H)
    x2 = jnp.minimum(jnp.maximum(nx + nw, 0.0), IMG_W)
    y2 = jnp.minimum(jnp.maximum(ny + nh, 0.0), IMG_H)
    area_b2 = (x2 - x1) * (y2 - y1)
    k2 = keep_mask(x1, y1, x2, y2, area_b2)
    gen[0:1, :] = x1 * k2
    gen[1:2, :] = y1 * k2
    gen[2:3, :] = x2 * k2
    gen[3:4, :] = y2 * k2
    gen[4:5, :] = k2


@jax.jit
def kernel(proposal_boxes, objectness_logits, gt_boxes, u_tx, u_ty, u_tw, u_th):
    pbT = jnp.pad(
        jnp.concatenate([proposal_boxes.T, objectness_logits[None, :]], axis=0),
        ((0, 0), (0, NPAD - N_PROP)),
    )
    g8 = jnp.concatenate(
        [
            jnp.repeat(gt_boxes.T, N_PER, axis=1),
            jnp.stack(
                [
                    u_tx.T.reshape(-1),
                    u_ty.T.reshape(-1),
                    u_tw.T.reshape(-1),
                    u_th.T.reshape(-1),
                ]
            ),
        ],
        axis=0,
    )
    main, gen = pl.pallas_call(
        _body,
        out_shape=[
            jax.ShapeDtypeStruct((5, NPAD), jnp.float32),
            jax.ShapeDtypeStruct((5, N_GEN), jnp.float32),
        ],
    )(pbT, gt_boxes, g8)
    return jnp.concatenate([main[:, :N_PROP], gen], axis=1).T


# speedup vs baseline: 5.3991x; 1.1976x over previous
import jax
import jax.numpy as jnp
import numpy as np
from jax.experimental import pallas as pl

ALPHA = 0.5
BETA = 0.3
T = 0.5
IMG_H = 1024.0
IMG_W = 1024.0
N_PROP = 20000
N_GT = 64
N_PER = 4
N_GEN = N_GT * N_PER
N_OUT = N_PROP + N_GEN
NPAD = 20480
CHUNK = 512

_LO = float(np.log(1.0 - BETA))
_HI = float(np.log(1.0 + BETA))


def _body(pbT, gtb, g8, main):
    gx1 = gtb[:, 0:1]
    gy1 = gtb[:, 1:2]
    gx2 = gtb[:, 2:3]
    gy2 = gtb[:, 3:4]
    ga = (gx2 - gx1) * (gy2 - gy1)

    def keep_mask(a1, b1, a2, b2, area_b):
        wx = jnp.maximum(jnp.minimum(gx2, a2) - jnp.maximum(gx1, a1), 0.0)
        wy = jnp.maximum(jnp.minimum(gy2, b2) - jnp.maximum(gy1, b1), 0.0)
        inter = wx * wy
        union = (ga + area_b) - inter
        iou = inter / jnp.maximum(union, 1e-9)
        m = jnp.max(iou, axis=0, keepdims=True)
        return jnp.where(m > T, 1.0, 0.0).astype(jnp.float32)

    for c in range(NPAD // CHUNK):
        s = c * CHUNK
        px1 = pbT[0:1, s : s + CHUNK]
        py1 = pbT[1:2, s : s + CHUNK]
        px2 = pbT[2:3, s : s + CHUNK]
        py2 = pbT[3:4, s : s + CHUNK]
        area_b = (px2 - px1) * (py2 - py1)
        k = keep_mask(px1, py1, px2, py2, area_b)
        vals = (px1 * k, py1 * k, px2 * k, py2 * k, pbT[4:5, s : s + CHUNK] * k)
        w = CHUNK if s + CHUNK <= N_PROP else N_PROP - s
        for row, v in enumerate(vals):
            main[row : row + 1, s : s + w] = v[:, :w]

    gg_x1 = g8[0:1, :]
    gg_y1 = g8[1:2, :]
    gg_x2 = g8[2:3, :]
    gg_y2 = g8[3:4, :]
    gw = gg_x2 - gg_x1
    gh = gg_y2 - gg_y1
    tx = (g8[4:5, :] - 0.5) * 2.0 * ALPHA
    ty = (g8[5:6, :] - 0.5) * 2.0 * ALPHA
    tw = _LO + g8[6:7, :] * (_HI - _LO)
    th = _LO + g8[7:8, :] * (_HI - _LO)
    nx = gg_x1 + gw * tx
    ny = gg_y1 + gh * ty
    nw = gw * jnp.exp(tw)
    nh = gh * jnp.exp(th)
    x1 = jnp.minimum(jnp.maximum(nx, 0.0), IMG_W)
    y1 = jnp.minimum(jnp.maximum(ny, 0.0), IMG_H)
    x2 = jnp.minimum(jnp.maximum(nx + nw, 0.0), IMG_W)
    y2 = jnp.minimum(jnp.maximum(ny + nh, 0.0), IMG_H)
    area_b2 = (x2 - x1) * (y2 - y1)
    k2 = keep_mask(x1, y1, x2, y2, area_b2)
    main[0:1, N_PROP:N_OUT] = x1 * k2
    main[1:2, N_PROP:N_OUT] = y1 * k2
    main[2:3, N_PROP:N_OUT] = x2 * k2
    main[3:4, N_PROP:N_OUT] = y2 * k2
    main[4:5, N_PROP:N_OUT] = k2


@jax.jit
def kernel(proposal_boxes, objectness_logits, gt_boxes, u_tx, u_ty, u_tw, u_th):
    pbT = jnp.pad(
        jnp.concatenate([proposal_boxes.T, objectness_logits[None, :]], axis=0),
        ((0, 0), (0, NPAD - N_PROP)),
    )
    g8 = jnp.concatenate(
        [
            jnp.repeat(gt_boxes.T, N_PER, axis=1),
            jnp.stack(
                [
                    u_tx.T.reshape(-1),
                    u_ty.T.reshape(-1),
                    u_tw.T.reshape(-1),
                    u_th.T.reshape(-1),
                ]
            ),
        ],
        axis=0,
    )
    main = pl.pallas_call(
        _body,
        out_shape=jax.ShapeDtypeStruct((5, N_OUT), jnp.float32),
    )(pbT, gt_boxes, g8)
    return main.T


# speedup vs baseline: 5.5615x; 1.0301x over previous
import jax
import jax.numpy as jnp
import numpy as np
from jax.experimental import pallas as pl

ALPHA = 0.5
BETA = 0.3
T = 0.5
IMG_H = 1024.0
IMG_W = 1024.0
N_PROP = 20000
N_GT = 64
N_PER = 4
N_GEN = N_GT * N_PER
N_OUT = N_PROP + N_GEN
NPAD = 20480
CHUNK = 256

_LO = float(np.log(1.0 - BETA))
_HI = float(np.log(1.0 + BETA))


def _body(pbT, gtb, g8, main):
    gx1 = gtb[:, 0:1]
    gy1 = gtb[:, 1:2]
    gx2 = gtb[:, 2:3]
    gy2 = gtb[:, 3:4]
    ga = (gx2 - gx1) * (gy2 - gy1)

    def keep_mask(a1, b1, a2, b2, area_b):
        wx = jnp.maximum(jnp.minimum(gx2, a2) - jnp.maximum(gx1, a1), 0.0)
        wy = jnp.maximum(jnp.minimum(gy2, b2) - jnp.maximum(gy1, b1), 0.0)
        inter = wx * wy
        union = (ga + area_b) - inter
        iou = inter / jnp.maximum(union, 1e-9)
        m = jnp.max(iou, axis=0, keepdims=True)
        return jnp.where(m > T, 1.0, 0.0).astype(jnp.float32)

    for c in range((N_PROP + CHUNK - 1) // CHUNK):
        s = c * CHUNK
        px1 = pbT[0:1, s : s + CHUNK]
        py1 = pbT[1:2, s : s + CHUNK]
        px2 = pbT[2:3, s : s + CHUNK]
        py2 = pbT[3:4, s : s + CHUNK]
        area_b = (px2 - px1) * (py2 - py1)
        k = keep_mask(px1, py1, px2, py2, area_b)
        vals = (px1 * k, py1 * k, px2 * k, py2 * k, pbT[4:5, s : s + CHUNK] * k)
        w = min(CHUNK, N_PROP - s)
        for row, v in enumerate(vals):
            main[row : row + 1, s : s + w] = v[:, :w]

    gg_x1 = g8[0:1, :]
    gg_y1 = g8[1:2, :]
    gg_x2 = g8[2:3, :]
    gg_y2 = g8[3:4, :]
    gw = gg_x2 - gg_x1
    gh = gg_y2 - gg_y1
    tx = (g8[4:5, :] - 0.5) * 2.0 * ALPHA
    ty = (g8[5:6, :] - 0.5) * 2.0 * ALPHA
    tw = _LO + g8[6:7, :] * (_HI - _LO)
    th = _LO + g8[7:8, :] * (_HI - _LO)
    nx = gg_x1 + gw * tx
    ny = gg_y1 + gh * ty
    nw = gw * jnp.exp(tw)
    nh = gh * jnp.exp(th)
    x1 = jnp.minimum(jnp.maximum(nx, 0.0), IMG_W)
    y1 = jnp.minimum(jnp.maximum(ny, 0.0), IMG_H)
    x2 = jnp.minimum(jnp.maximum(nx + nw, 0.0), IMG_W)
    y2 = jnp.minimum(jnp.maximum(ny + nh, 0.0), IMG_H)
    area_b2 = (x2 - x1) * (y2 - y1)
    k2 = keep_mask(x1, y1, x2, y2, area_b2)
    main[0:1, N_PROP:N_OUT] = x1 * k2
    main[1:2, N_PROP:N_OUT] = y1 * k2
    main[2:3, N_PROP:N_OUT] = x2 * k2
    main[3:4, N_PROP:N_OUT] = y2 * k2
    main[4:5, N_PROP:N_OUT] = k2


@jax.jit
def kernel(proposal_boxes, objectness_logits, gt_boxes, u_tx, u_ty, u_tw, u_th):
    pbT = jnp.pad(
        jnp.concatenate([proposal_boxes.T, objectness_logits[None, :]], axis=0),
        ((0, 0), (0, NPAD - N_PROP)),
    )
    g8 = jnp.concatenate(
        [
            jnp.repeat(gt_boxes.T, N_PER, axis=1),
            jnp.stack(
                [
                    u_tx.T.reshape(-1),
                    u_ty.T.reshape(-1),
                    u_tw.T.reshape(-1),
                    u_th.T.reshape(-1),
                ]
            ),
        ],
        axis=0,
    )
    main = pl.pallas_call(
        _body,
        out_shape=jax.ShapeDtypeStruct((5, N_OUT), jnp.float32),
    )(pbT, gt_boxes, g8)
    return main.T
